# baseline (speedup 1.0000x reference)
import jax
import jax.numpy as jnp
from jax import lax
from jax.experimental import pallas as pl
from jax.experimental.pallas import tpu as pltpu
from jax.experimental.pallas import tpu_sc as plsc

DIM = 512
DIM_KQ = 8
DIM_V = 16
TBL = 2 * DIM - 1
LANES = 16
NC, NS = 2, 16
N_BANK = 8
BW_SC = 640
BW_TC = 1024
N_GRP = DIM // LANES


def _sc_body(rel_hbm, q_hbm, k_hbm, tbl_v, c3_v, sem):
    wid = lax.axis_index("s") * NC + lax.axis_index("c")
    ch = lax.shift_right_logical(wid, 1)
    p = lax.bitwise_and(wid, 1)
    pltpu.sync_copy(rel_hbm.at[ch], tbl_v)
    iota = lax.iota(jnp.int32, LANES)

    qo = 256 * (1 - p)

    @pl.loop(0, N_BANK)
    def _bank(l):
        @plsc.parallel_loop(0, BW_SC // LANES, 1, unroll=2)
        def _chunk(k):
            for u in range(LANES):
                idx = (1007 + u - 16 * k) - 16 * l - qo - iota
                vals = plsc.load_gather(tbl_v, [jnp.clip(idx, 0, TBL - 1)])
                c3_v[l, u, pl.ds(k * LANES, LANES)] = vals

        for h_rel in range(2):
            n = 31 - l - 8 * (2 * (1 - p) + h_rel)
            src = c3_v.at[l, :, pl.ds(128 * h_rel, DIM)]
            rows = pl.ds(16 * n, 16)

            @pl.when(ch < DIM_KQ)
            def _():
                pltpu.async_copy(src, q_hbm.at[ch, rows], sem)

            @pl.when(ch >= DIM_KQ)
            def _():
                pltpu.async_copy(src, k_hbm.at[ch - DIM_KQ, rows], sem)

    @pl.loop(0, LANES)
    def _drain(n):
        pltpu.make_async_copy(
            c3_v.at[0, :, pl.ds(0, DIM)], q_hbm.at[0, pl.ds(0, 16)], sem
        ).wait()


def _sc_call(relative):
    return pl.kernel(
        _sc_body,
        out_type=(
            jax.ShapeDtypeStruct((DIM_KQ, DIM, DIM), jnp.float32),
            jax.ShapeDtypeStruct((DIM_KQ, DIM, DIM), jnp.float32),
        ),
        mesh=plsc.VectorSubcoreMesh(
            core_axis_name="c", subcore_axis_name="s",
            num_cores=NC, num_subcores=NS,
        ),
        scratch_types=[
            pltpu.VMEM((TBL,), jnp.float32),
            pltpu.VMEM((N_BANK, LANES, BW_SC), jnp.float32),
            pltpu.SemaphoreType.DMA,
        ],
        compiler_params=pltpu.CompilerParams(
            needs_layout_passes=False, skip_device_barrier=True,
        ),
    )(relative)


TC_CPB = 4


def _tc_body(rel_ref, o_ref):
    for cc in range(TC_CPB):
        flip = rel_ref[cc]
        ext = jnp.concatenate([flip, flip[:, -1:]], axis=1)
        x16 = jnp.broadcast_to(ext, (LANES, BW_TC))
        for l in range(N_BANK):
            bank = pltpu.roll(
                x16, BW_TC - (16 * l + 15), 1, stride=1, stride_axis=0)
            for h in range(4):
                n = 31 - l - 8 * h
                o_ref[cc, pl.ds(16 * n, 16), :] = (
                    bank[:, 128 * h:128 * h + DIM])


def _tc_call(relative):
    return pl.pallas_call(
        _tc_body,
        grid=(DIM_V // TC_CPB,),
        in_specs=[pl.BlockSpec((TC_CPB, 1, TBL), lambda i: (i, 0, 0))],
        out_specs=pl.BlockSpec((TC_CPB, DIM, DIM), lambda i: (i, 0, 0)),
        out_shape=jax.ShapeDtypeStruct((DIM_V, DIM, DIM), jnp.float32),
    )(relative)


def kernel(relative):
    q, k = _sc_call(relative)
    v = _tc_call(relative[2 * DIM_KQ:, None, ::-1])
    return q, k, v


# speedup vs baseline: 1.0060x; 1.0060x over previous
import jax
import jax.numpy as jnp
from jax import lax
from jax.experimental import pallas as pl
from jax.experimental.pallas import tpu as pltpu
from jax.experimental.pallas import tpu_sc as plsc

DIM = 512
DIM_KQ = 8
DIM_V = 16
TBL = 2 * DIM - 1
LANES = 16
NC, NS = 2, 16
N_BANK = 8
BW_SC = 640
BW_TC = 1024
N_GRP = DIM // LANES


def _sc_body(rel_hbm, q_hbm, k_hbm, tbl_v, c3_v, sem):
    wid = lax.axis_index("s") * NC + lax.axis_index("c")
    ch = lax.shift_right_logical(wid, 1)
    p = lax.bitwise_and(wid, 1)
    pltpu.sync_copy(rel_hbm.at[ch], tbl_v)
    iota = lax.iota(jnp.int32, LANES)

    qo = 256 * (1 - p)

    @pl.loop(0, N_BANK)
    def _bank(l):
        @plsc.parallel_loop(0, BW_SC // LANES, 1, unroll=2)
        def _chunk(k):
            for u in range(LANES):
                idx = (1007 + u - 16 * k) - 16 * l - qo - iota
                vals = plsc.load_gather(tbl_v, [jnp.clip(idx, 0, TBL - 1)])
                c3_v[l, u, pl.ds(k * LANES, LANES)] = vals

        for h_rel in range(2):
            n = 31 - l - 8 * (2 * (1 - p) + h_rel)
            src = c3_v.at[l, :, pl.ds(128 * h_rel, DIM)]
            rows = pl.ds(16 * n, 16)

            @pl.when(ch < DIM_KQ)
            def _():
                pltpu.async_copy(src, q_hbm.at[ch, rows], sem)

            @pl.when(ch >= DIM_KQ)
            def _():
                pltpu.async_copy(src, k_hbm.at[ch - DIM_KQ, rows], sem)

    @pl.loop(0, LANES)
    def _drain(n):
        pltpu.make_async_copy(
            c3_v.at[0, :, pl.ds(0, DIM)], q_hbm.at[0, pl.ds(0, 16)], sem
        ).wait()


def _sc_call(relative):
    return pl.kernel(
        _sc_body,
        out_type=(
            jax.ShapeDtypeStruct((DIM_KQ, DIM, DIM), jnp.float32),
            jax.ShapeDtypeStruct((DIM_KQ, DIM, DIM), jnp.float32),
        ),
        mesh=plsc.VectorSubcoreMesh(
            core_axis_name="c", subcore_axis_name="s",
            num_cores=NC, num_subcores=NS,
        ),
        scratch_types=[
            pltpu.VMEM((TBL,), jnp.float32),
            pltpu.VMEM((N_BANK, LANES, BW_SC), jnp.float32),
            pltpu.SemaphoreType.DMA,
        ],
        compiler_params=pltpu.CompilerParams(
            needs_layout_passes=False, skip_device_barrier=True,
        ),
    )(relative)


TC_CPB = 2


def _tc_body(rel_ref, o_ref):
    for cc in range(TC_CPB):
        flip = rel_ref[cc]
        ext = jnp.concatenate([flip, flip[:, -1:]], axis=1)
        x16 = jnp.broadcast_to(ext, (LANES, BW_TC))
        for l in range(N_BANK):
            bank = pltpu.roll(
                x16, BW_TC - (16 * l + 15), 1, stride=1, stride_axis=0)
            for h in range(4):
                n = 31 - l - 8 * h
                o_ref[cc, pl.ds(16 * n, 16), :] = (
                    bank[:, 128 * h:128 * h + DIM])


def _tc_call(relative):
    return pl.pallas_call(
        _tc_body,
        grid=(DIM_V // TC_CPB,),
        in_specs=[pl.BlockSpec((TC_CPB, 1, TBL), lambda i: (i, 0, 0))],
        out_specs=pl.BlockSpec((TC_CPB, DIM, DIM), lambda i: (i, 0, 0)),
        out_shape=jax.ShapeDtypeStruct((DIM_V, DIM, DIM), jnp.float32),
    )(relative)


def kernel(relative):
    q, k = _sc_call(relative)
    v = _tc_call(relative[2 * DIM_KQ:, None, ::-1])
    return q, k, v


# speedup vs baseline: 1.0079x; 1.0019x over previous
import jax
import jax.numpy as jnp
from jax import lax
from jax.experimental import pallas as pl
from jax.experimental.pallas import tpu as pltpu
from jax.experimental.pallas import tpu_sc as plsc

DIM = 512
DIM_KQ = 8
DIM_V = 16
TBL = 2 * DIM - 1
LANES = 16
NC, NS = 2, 16
N_BANK = 8
BW_SC = 640
BW_TC = 1024
N_GRP = DIM // LANES


def _sc_body(rel_hbm, rev_hbm, q_hbm, k_hbm, tbl_v, c3_v, sem):
    del rev_hbm
    wid = lax.axis_index("s") * NC + lax.axis_index("c")
    ch = lax.shift_right_logical(wid, 1)
    p = lax.bitwise_and(wid, 1)
    pltpu.sync_copy(rel_hbm.at[ch], tbl_v)
    iota = lax.iota(jnp.int32, LANES)

    qo = 256 * (1 - p)

    @pl.loop(0, N_BANK)
    def _bank(l):
        @plsc.parallel_loop(0, BW_SC // LANES, 1, unroll=2)
        def _chunk(k):
            for u in range(LANES):
                idx = (1007 + u - 16 * k) - 16 * l - qo - iota
                vals = plsc.load_gather(tbl_v, [jnp.clip(idx, 0, TBL - 1)])
                c3_v[l, u, pl.ds(k * LANES, LANES)] = vals

        for h_rel in range(2):
            n = 31 - l - 8 * (2 * (1 - p) + h_rel)
            src = c3_v.at[l, :, pl.ds(128 * h_rel, DIM)]
            rows = pl.ds(16 * n, 16)

            @pl.when(ch < DIM_KQ)
            def _():
                pltpu.async_copy(src, q_hbm.at[ch, rows], sem)

            @pl.when(ch >= DIM_KQ)
            def _():
                pltpu.async_copy(src, k_hbm.at[ch - DIM_KQ, rows], sem)

    @pl.loop(0, LANES)
    def _drain(n):
        pltpu.make_async_copy(
            c3_v.at[0, :, pl.ds(0, DIM)], q_hbm.at[0, pl.ds(0, 16)], sem
        ).wait()


def _sc_call(relative, rev):
    return pl.kernel(
        _sc_body,
        out_type=(
            jax.ShapeDtypeStruct((DIM_KQ, DIM, DIM), jnp.float32),
            jax.ShapeDtypeStruct((DIM_KQ, DIM, DIM), jnp.float32),
        ),
        mesh=plsc.VectorSubcoreMesh(
            core_axis_name="c", subcore_axis_name="s",
            num_cores=NC, num_subcores=NS,
        ),
        scratch_types=[
            pltpu.VMEM((TBL,), jnp.float32),
            pltpu.VMEM((N_BANK, LANES, BW_SC), jnp.float32),
            pltpu.SemaphoreType.DMA,
        ],
        compiler_params=pltpu.CompilerParams(
            needs_layout_passes=False, skip_device_barrier=True,
        ),
    )(relative, rev)


TC_CPB = 2


def _tc_body(rel_ref, o_ref):
    for cc in range(TC_CPB):
        flip = rel_ref[cc]
        ext = jnp.concatenate([flip, flip[:, -1:]], axis=1)
        x16 = jnp.broadcast_to(ext, (LANES, BW_TC))
        for l in range(N_BANK):
            bank = pltpu.roll(
                x16, BW_TC - (16 * l + 15), 1, stride=1, stride_axis=0)
            for h in range(4):
                n = 31 - l - 8 * h
                o_ref[cc, pl.ds(16 * n, 16), :] = (
                    bank[:, 128 * h:128 * h + DIM])


def _tc_call(relative):
    return pl.pallas_call(
        _tc_body,
        grid=(DIM_V // TC_CPB,),
        in_specs=[pl.BlockSpec((TC_CPB, 1, TBL), lambda i: (i, 0, 0))],
        out_specs=pl.BlockSpec((TC_CPB, DIM, DIM), lambda i: (i, 0, 0)),
        out_shape=jax.ShapeDtypeStruct((DIM_V, DIM, DIM), jnp.float32),
    )(relative)


def kernel(relative):
    rev = relative[2 * DIM_KQ:, None, ::-1]
    q, k = _sc_call(relative, rev)
    v = _tc_call(rev)
    return q, k, v
